# Initial kernel scaffold; baseline (speedup 1.0000x reference)
import jax
import jax.numpy as jnp
from jax.experimental import pallas as pl


def kernel(bbox, Wx, Wy, Wh, Ww):
    raise NotImplementedError("write your pallas kernel here")


# speedup vs baseline: 2.3726x; 2.3726x over previous
import functools

import jax
import jax.numpy as jnp
from jax import lax
from jax.experimental import pallas as pl
from jax.experimental.pallas import tpu as pltpu
from jax.experimental.pallas import tpu_sc as plsc

B, S = 64, 512
N = B * S
D = 256
NQ = 4

_info = plsc.get_sparse_core_info()
NC, NS = _info.num_cores, _info.num_subcores
NW = NC * NS
B_PER_W = N // NW
CHUNK = 64
N_CHUNKS = B_PER_W // CHUNK

_mesh = plsc.VectorSubcoreMesh(core_axis_name="c", subcore_axis_name="s")


@functools.partial(
    pl.kernel,
    mesh=_mesh,
    out_type=jax.ShapeDtypeStruct((N, NQ, D), jnp.float32),
    scratch_types=[
        pltpu.VMEM((B_PER_W,), jnp.int32),
        pltpu.VMEM((B_PER_W,), jnp.int32),
        pltpu.VMEM((B_PER_W,), jnp.int32),
        pltpu.VMEM((B_PER_W,), jnp.int32),
        pltpu.VMEM((CHUNK, D), jnp.float32),
        pltpu.VMEM((CHUNK, D), jnp.float32),
        pltpu.VMEM((CHUNK, D), jnp.float32),
        pltpu.VMEM((CHUNK, D), jnp.float32),
        pltpu.SemaphoreType.DMA,
    ],
)
def _gather_kernel(i0, i1, i2, i3, wx, wy, wh, ww, out,
                   v0, v1, v2, v3, r0, r1, r2, r3, sem):
    wid = lax.axis_index("s") * NC + lax.axis_index("c")
    base0 = pl.multiple_of(wid * B_PER_W, B_PER_W)

    idx_refs = (v0, v1, v2, v3)
    row_refs = (r0, r1, r2, r3)
    tables = (wx, wy, wh, ww)
    idx_hbm = (i0, i1, i3, i2)

    for q in range(NQ):
        pltpu.sync_copy(idx_hbm[q].at[pl.ds(base0, B_PER_W)], idx_refs[q])

    def body(ci, carry):
        off = pl.multiple_of(ci * CHUNK, CHUNK)
        base = base0 + off
        copies = [
            pltpu.async_copy(
                tables[q].at[idx_refs[q].at[pl.ds(off, CHUNK)]],
                row_refs[q], sem)
            for q in range(NQ)
        ]
        for c in copies:
            c.wait()
        for q in range(NQ):
            pltpu.sync_copy(row_refs[q], out.at[pl.ds(base, CHUNK), q])
        return carry

    lax.fori_loop(0, N_CHUNKS, body, 0)


def kernel(bbox, Wx, Wy, Wh, Ww):
    cols = bbox.reshape(N, NQ)
    i0 = cols[:, 0]
    i1 = cols[:, 1]
    i2 = cols[:, 2]
    i3 = cols[:, 3]
    out = _gather_kernel(i0, i1, i2, i3, Wx, Wy, Wh, Ww)
    return out.reshape(B, S, NQ * D)


# speedup vs baseline: 2.4274x; 1.0231x over previous
import functools

import jax
import jax.numpy as jnp
from jax import lax
from jax.experimental import pallas as pl
from jax.experimental.pallas import tpu as pltpu
from jax.experimental.pallas import tpu_sc as plsc

B, S = 64, 512
N = B * S
D = 256
NQ = 4

_info = plsc.get_sparse_core_info()
NC, NS = _info.num_cores, _info.num_subcores
NW = NC * NS
B_PER_W = N // NW
CHUNK = 32
NBUF = 2
N_CHUNKS = B_PER_W // CHUNK
N_GROUPS = N_CHUNKS // NBUF

_mesh = plsc.VectorSubcoreMesh(core_axis_name="c", subcore_axis_name="s")


@functools.partial(
    pl.kernel,
    mesh=_mesh,
    out_type=jax.ShapeDtypeStruct((N, NQ, D), jnp.float32),
    scratch_types=(
        [pltpu.VMEM((B_PER_W,), jnp.int32) for _ in range(NQ)]
        + [pltpu.VMEM((CHUNK, D), jnp.float32) for _ in range(NBUF * NQ)]
        + [pltpu.SemaphoreType.DMA for _ in range(2 * NBUF)]
    ),
)
def _gather_kernel(i0, i1, i2, i3, wx, wy, wh, ww, out, *scratch):
    idx_refs = scratch[:NQ]
    rows = tuple(
        scratch[NQ + b * NQ: NQ + (b + 1) * NQ] for b in range(NBUF))
    sem_g = scratch[NQ + NBUF * NQ: NQ + NBUF * NQ + NBUF]
    sem_s = scratch[NQ + NBUF * NQ + NBUF:]

    tables = (wx, wy, wh, ww)
    idx_hbm = (i0, i1, i3, i2)

    wid = lax.axis_index("s") * NC + lax.axis_index("c")
    base0 = pl.multiple_of(wid * B_PER_W, B_PER_W)

    for q in range(NQ):
        pltpu.sync_copy(idx_hbm[q].at[pl.ds(base0, B_PER_W)], idx_refs[q])

    def group(g, carry):
        goff = pl.multiple_of(g * (NBUF * CHUNK), NBUF * CHUNK)
        for b in range(NBUF):
            off = goff + b * CHUNK

            @pl.when(g > 0)
            def _drain_prev_stores():
                for q in range(NQ):
                    pltpu.make_async_copy(
                        rows[b][q], out.at[pl.ds(base0, CHUNK), q],
                        sem_s[b]).wait()

            for q in range(NQ):
                pltpu.async_copy(
                    tables[q].at[idx_refs[q].at[pl.ds(off, CHUNK)]],
                    rows[b][q], sem_g[b])
        for b in range(NBUF):
            base = base0 + goff + b * CHUNK
            for q in range(NQ):
                pltpu.make_async_copy(
                    tables[q].at[idx_refs[q].at[pl.ds(0, CHUNK)]],
                    rows[b][q], sem_g[b]).wait()
            for q in range(NQ):
                pltpu.async_copy(
                    rows[b][q], out.at[pl.ds(base, CHUNK), q], sem_s[b])
        return carry

    lax.fori_loop(0, N_GROUPS, group, 0)

    for b in range(NBUF):
        for q in range(NQ):
            pltpu.make_async_copy(
                rows[b][q], out.at[pl.ds(base0, CHUNK), q], sem_s[b]).wait()


def kernel(bbox, Wx, Wy, Wh, Ww):
    cols = bbox.reshape(N, NQ)
    out = _gather_kernel(cols[:, 0], cols[:, 1], cols[:, 2], cols[:, 3],
                         Wx, Wy, Wh, Ww)
    return out.reshape(B, S, NQ * D)


# speedup vs baseline: 5.2797x; 2.1750x over previous
import functools

import jax
import jax.numpy as jnp
from jax import lax
from jax.experimental import pallas as pl
from jax.experimental.pallas import tpu as pltpu
from jax.experimental.pallas import tpu_sc as plsc

B, S = 64, 512
N = B * S
D = 256
NQ = 4

_info = plsc.get_sparse_core_info()
NC, NS = _info.num_cores, _info.num_subcores
NW = NC * NS
B_PER_W = N // NW
CHUNK = 32
NBUF = 2
N_CHUNKS = B_PER_W // CHUNK
N_GROUPS = N_CHUNKS // NBUF

_mesh = plsc.VectorSubcoreMesh(core_axis_name="c", subcore_axis_name="s")


@functools.partial(
    pl.kernel,
    mesh=_mesh,
    out_type=jax.ShapeDtypeStruct((N, NQ * D), jnp.float32),
    scratch_types=(
        [pltpu.VMEM((B_PER_W,), jnp.int32) for _ in range(NQ)]
        + [pltpu.VMEM((CHUNK, D), jnp.float32) for _ in range(NBUF * NQ)]
        + [pltpu.SemaphoreType.DMA for _ in range(2 * NBUF)]
    ),
)
def _gather_kernel(i0, i1, i2, i3, wx, wy, wh, ww, out, *scratch):
    idx_refs = scratch[:NQ]
    rows = tuple(
        scratch[NQ + b * NQ: NQ + (b + 1) * NQ] for b in range(NBUF))
    sem_g = scratch[NQ + NBUF * NQ: NQ + NBUF * NQ + NBUF]
    sem_s = scratch[NQ + NBUF * NQ + NBUF:]

    tables = (wx, wy, wh, ww)
    idx_hbm = (i0, i1, i3, i2)

    wid = lax.axis_index("s") * NC + lax.axis_index("c")
    base0 = pl.multiple_of(wid * B_PER_W, B_PER_W)

    for q in range(NQ):
        pltpu.sync_copy(idx_hbm[q].at[pl.ds(base0, B_PER_W)], idx_refs[q])

    def group(g, carry):
        goff = pl.multiple_of(g * (NBUF * CHUNK), NBUF * CHUNK)
        for b in range(NBUF):
            off = goff + b * CHUNK

            @pl.when(g > 0)
            def _drain_prev_stores():
                for q in range(NQ):
                    pltpu.make_async_copy(
                        rows[b][q],
                        out.at[pl.ds(base0, CHUNK), pl.ds(q * D, D)],
                        sem_s[b]).wait()

            for q in range(NQ):
                pltpu.async_copy(
                    tables[q].at[idx_refs[q].at[pl.ds(off, CHUNK)]],
                    rows[b][q], sem_g[b])
        for b in range(NBUF):
            base = base0 + goff + b * CHUNK
            for q in range(NQ):
                pltpu.make_async_copy(
                    tables[q].at[idx_refs[q].at[pl.ds(0, CHUNK)]],
                    rows[b][q], sem_g[b]).wait()
            for q in range(NQ):
                pltpu.async_copy(
                    rows[b][q],
                    out.at[pl.ds(base, CHUNK), pl.ds(q * D, D)], sem_s[b])
        return carry

    lax.fori_loop(0, N_GROUPS, group, 0)

    for b in range(NBUF):
        for q in range(NQ):
            pltpu.make_async_copy(
                rows[b][q],
                out.at[pl.ds(base0, CHUNK), pl.ds(q * D, D)],
                sem_s[b]).wait()


def kernel(bbox, Wx, Wy, Wh, Ww):
    cols = bbox.reshape(N, NQ)
    out = _gather_kernel(cols[:, 0], cols[:, 1], cols[:, 2], cols[:, 3],
                         Wx, Wy, Wh, Ww)
    return out.reshape(B, S, NQ * D)
